# Initial kernel scaffold; baseline (speedup 1.0000x reference)
import jax
import jax.numpy as jnp
from jax.experimental import pallas as pl


def kernel(x, edge_index, W_src1, W_dst1, att_src1, att_dst1, b1, Wl1, bl1, W_src2, W_dst2, att_src2, att_dst2, b2, Wl2, bl2):
    raise NotImplementedError("write your pallas kernel here")


# speedup vs baseline: 27.6591x; 27.6591x over previous
import jax
import jax.numpy as jnp
from jax import lax
from jax.experimental import pallas as pl
from jax.experimental.pallas import tpu as pltpu
from jax.experimental.pallas import tpu_sc as plsc

NN = 10000
EE = 320000
FD = 128
NC, NS, LN = 2, 16, 16
NW = NC * NS
EPT = EE // NW
CK = 80
NCHUNK = EPT // CK
NP = 10240
RPT = NP // NS
EPS = 1e-16

RB = 400
GRID = NN // RB


def _pre_body(x_ref, ws_ref, atts_ref, wd_ref, attd_ref, wl_ref, bl_ref,
              xs_ref, as_ref, ad_ref, xl_ref):
    xb = x_ref[...]
    xs = jnp.dot(xb, ws_ref[...], preferred_element_type=jnp.float32)
    xs_ref[...] = xs
    as_ref[...] = jnp.dot(xs, atts_ref[...], preferred_element_type=jnp.float32)
    wd = jnp.dot(wd_ref[...], attd_ref[...], preferred_element_type=jnp.float32)
    ad_ref[...] = jnp.dot(xb, wd, preferred_element_type=jnp.float32)
    xl_ref[...] = jnp.dot(xb, wl_ref[...], preferred_element_type=jnp.float32) + bl_ref[...]


def _dense_pre(x, W_src, att_src, W_dst, att_dst, Wl, bl):
    full = lambda s: pl.BlockSpec(s, lambda i: (0, 0))
    blk = lambda s: pl.BlockSpec(s, lambda i: (i, 0))
    return pl.pallas_call(
        _pre_body,
        grid=(GRID,),
        in_specs=[blk((RB, FD)), full((FD, FD)), full((FD, 1)),
                  full((FD, FD)), full((FD, 1)), full((FD, FD)), full((1, FD))],
        out_specs=[blk((RB, FD)), blk((RB, 1)), blk((RB, 1)), blk((RB, FD))],
        out_shape=[jax.ShapeDtypeStruct((NN, FD), jnp.float32),
                   jax.ShapeDtypeStruct((NN, 1), jnp.float32),
                   jax.ShapeDtypeStruct((NN, 1), jnp.float32),
                   jax.ShapeDtypeStruct((NN, FD), jnp.float32)],
    )(x, W_src, att_src.reshape(FD, 1), W_dst, att_dst.reshape(FD, 1),
      Wl, bl.reshape(1, FD))


def _mid_body(n0_ref, n1_ref, d0_ref, d1_ref, b_ref, xl_ref,
              ws_ref, atts_ref, wd_ref, attd_ref, wl_ref, bl_ref,
              xs_ref, as_ref, ad_ref, xl2_ref):
    num = n0_ref[...] + n1_ref[...]
    den = d0_ref[...] + d1_ref[...]
    h = num / (den + EPS) + b_ref[...] + xl_ref[...]
    h = jnp.maximum(h, 0.0)
    xs = jnp.dot(h, ws_ref[...], preferred_element_type=jnp.float32)
    xs_ref[...] = xs
    as_ref[...] = jnp.dot(xs, atts_ref[...], preferred_element_type=jnp.float32)
    wd = jnp.dot(wd_ref[...], attd_ref[...], preferred_element_type=jnp.float32)
    ad_ref[...] = jnp.dot(h, wd, preferred_element_type=jnp.float32)
    xl2_ref[...] = jnp.dot(h, wl_ref[...], preferred_element_type=jnp.float32) + bl_ref[...]


def _dense_mid(num, den, b, xl, W_src, att_src, W_dst, att_dst, Wl, bl):
    full = lambda s: pl.BlockSpec(s, lambda i: (0, 0))
    blk = lambda s: pl.BlockSpec(s, lambda i: (i, 0))
    return pl.pallas_call(
        _mid_body,
        grid=(GRID,),
        in_specs=[blk((RB, FD)), blk((RB, FD)), blk((RB, 1)), blk((RB, 1)),
                  full((1, FD)), blk((RB, FD)),
                  full((FD, FD)), full((FD, 1)), full((FD, FD)), full((FD, 1)),
                  full((FD, FD)), full((1, FD))],
        out_specs=[blk((RB, FD)), blk((RB, 1)), blk((RB, 1)), blk((RB, FD))],
        out_shape=[jax.ShapeDtypeStruct((NN, FD), jnp.float32),
                   jax.ShapeDtypeStruct((NN, 1), jnp.float32),
                   jax.ShapeDtypeStruct((NN, 1), jnp.float32),
                   jax.ShapeDtypeStruct((NN, FD), jnp.float32)],
    )(num[0], num[1], den[0].reshape(NP, 1), den[1].reshape(NP, 1),
      b.reshape(1, FD), xl,
      W_src, att_src.reshape(FD, 1), W_dst, att_dst.reshape(FD, 1),
      Wl, bl.reshape(1, FD))


def _fin_body(n0_ref, n1_ref, d0_ref, d1_ref, b_ref, xl_ref, out_ref):
    num = n0_ref[...] + n1_ref[...]
    den = d0_ref[...] + d1_ref[...]
    out_ref[...] = num / (den + EPS) + b_ref[...] + xl_ref[...]


def _dense_fin(num, den, b, xl):
    full = lambda s: pl.BlockSpec(s, lambda i: (0, 0))
    blk = lambda s: pl.BlockSpec(s, lambda i: (i, 0))
    return pl.pallas_call(
        _fin_body,
        grid=(GRID,),
        in_specs=[blk((RB, FD)), blk((RB, FD)), blk((RB, 1)), blk((RB, 1)),
                  full((1, FD)), blk((RB, FD))],
        out_specs=blk((RB, FD)),
        out_shape=jax.ShapeDtypeStruct((NN, FD), jnp.float32),
    )(num[0], num[1], den[0].reshape(NP, 1), den[1].reshape(NP, 1),
      b.reshape(1, FD), xl)


def _edge_body(xs_hbm, as_hbm, ad_hbm, src_hbm, dst_hbm,
               num_out, den_out,
               src_v, dst_v, rows_v, e2_v, asg_v, adg_v, zden_v,
               as_sh, ad_sh, acc_num, acc_den, sem):
    c = lax.axis_index("c")
    s = lax.axis_index("s")
    wid = c * NS + s
    base = s * RPT

    z16 = jnp.zeros((LN,), jnp.float32)

    @pl.loop(0, CK)
    def _zr(i):
        for cc in range(FD // LN):
            rows_v[i, pl.ds(cc * LN, LN)] = z16

    @pl.loop(0, RPT // LN)
    def _zd(i):
        zden_v[pl.ds(i * LN, LN)] = z16

    for k in range(RPT // CK):
        pltpu.sync_copy(rows_v, acc_num.at[pl.ds(base + k * CK, CK)])
    pltpu.sync_copy(zden_v, acc_den.at[pl.ds(base, RPT)])

    @pl.when(s == 0)
    def _stage():
        pltpu.sync_copy(as_hbm, as_sh)
        pltpu.sync_copy(ad_hbm, ad_sh)

    pltpu.sync_copy(src_hbm.at[wid], src_v)
    pltpu.sync_copy(dst_hbm.at[wid], dst_v)

    plsc.subcore_barrier()

    @pl.loop(0, NCHUNK)
    def _chunk(ci):
        pltpu.sync_copy(as_sh.at[src_v.at[ci]], asg_v)
        pltpu.sync_copy(ad_sh.at[dst_v.at[ci]], adg_v)
        e2s = []
        for g in range(CK // LN):
            e = asg_v[pl.ds(g * LN, LN)] + adg_v[pl.ds(g * LN, LN)]
            e = jnp.where(e > 0.0, e, e * 0.2)
            e2 = jnp.exp(e)
            e2_v[pl.ds(g * LN, LN)] = e2
            e2s.append(e2)

        pltpu.async_copy(xs_hbm.at[src_v.at[ci]], rows_v, sem).wait()

        for g in range(CK // LN):
            ev = e2s[g]
            for j in range(LN):
                sv = ev[j]
                r = g * LN + j
                for cc in range(FD // LN):
                    rows_v[r, pl.ds(cc * LN, LN)] = rows_v[r, pl.ds(cc * LN, LN)] * sv

        pltpu.sync_copy(rows_v, acc_num.at[dst_v.at[ci]], add=True)
        pltpu.sync_copy(e2_v, acc_den.at[dst_v.at[ci]], add=True)

    plsc.subcore_barrier()

    pltpu.sync_copy(acc_num.at[pl.ds(base, RPT)], num_out.at[c, pl.ds(base, RPT)])
    pltpu.sync_copy(acc_den.at[pl.ds(base, RPT)], den_out.at[c, pl.ds(base, RPT)])


def _gat_edges(xs, a_s, a_d, src_r, dst_r):
    mesh = plsc.VectorSubcoreMesh(core_axis_name="c", subcore_axis_name="s",
                                  num_cores=NC, num_subcores=NS)
    return pl.kernel(
        _edge_body,
        out_type=(jax.ShapeDtypeStruct((NC, NP, FD), jnp.float32),
                  jax.ShapeDtypeStruct((NC, NP), jnp.float32)),
        mesh=mesh,
        compiler_params=pltpu.CompilerParams(needs_layout_passes=False,
                                             use_tc_tiling_on_sc=False),
        scratch_types=[
            pltpu.VMEM((NCHUNK, CK), jnp.int32),
            pltpu.VMEM((NCHUNK, CK), jnp.int32),
            pltpu.VMEM((CK, FD), jnp.float32),
            pltpu.VMEM((CK,), jnp.float32),
            pltpu.VMEM((CK,), jnp.float32),
            pltpu.VMEM((CK,), jnp.float32),
            pltpu.VMEM((RPT,), jnp.float32),
            pltpu.VMEM_SHARED((NN,), jnp.float32),
            pltpu.VMEM_SHARED((NN,), jnp.float32),
            pltpu.VMEM_SHARED((NP, FD), jnp.float32),
            pltpu.VMEM_SHARED((NP,), jnp.float32),
            pltpu.SemaphoreType.DMA,
        ],
    )(xs, a_s, a_d, src_r, dst_r)


def kernel(x, edge_index, W_src1, W_dst1, att_src1, att_dst1, b1, Wl1, bl1,
           W_src2, W_dst2, att_src2, att_dst2, b2, Wl2, bl2):
    src_r = edge_index[0].reshape(NW, NCHUNK, CK)
    dst_r = edge_index[1].reshape(NW, NCHUNK, CK)

    xs1, a_s1, a_d1, xl1 = _dense_pre(x, W_src1, att_src1, W_dst1, att_dst1,
                                      Wl1, bl1)
    num1, den1 = _gat_edges(xs1, a_s1.reshape(NN), a_d1.reshape(NN),
                            src_r, dst_r)
    xs2, a_s2, a_d2, xl2 = _dense_mid(num1, den1, b1, xl1, W_src2, att_src2,
                                      W_dst2, att_dst2, Wl2, bl2)
    num2, den2 = _gat_edges(xs2, a_s2.reshape(NN), a_d2.reshape(NN),
                            src_r, dst_r)
    return _dense_fin(num2, den2, b2, xl2)


# speedup vs baseline: 32.5357x; 1.1763x over previous
import jax
import jax.numpy as jnp
from jax import lax
from jax.experimental import pallas as pl
from jax.experimental.pallas import tpu as pltpu
from jax.experimental.pallas import tpu_sc as plsc

NN = 10000
EE = 320000
FD = 128
NC, NS, LN = 2, 16, 16
NW = NC * NS
EPT = EE // NW
CK = 80
NCHUNK = EPT // CK
NP = 10240
RPT = NP // NS
EPS = 1e-16

RB = 400
GRID = NN // RB


def _pre_body(x_ref, ws_ref, atts_ref, wd_ref, attd_ref, wl_ref, bl_ref,
              xs_ref, as_ref, ad_ref, xl_ref):
    xb = x_ref[...]
    xs = jnp.dot(xb, ws_ref[...], preferred_element_type=jnp.float32)
    xs_ref[...] = xs
    as_ref[...] = jnp.dot(xs, atts_ref[...], preferred_element_type=jnp.float32)
    wd = jnp.dot(wd_ref[...], attd_ref[...], preferred_element_type=jnp.float32)
    ad_ref[...] = jnp.dot(xb, wd, preferred_element_type=jnp.float32)
    xl_ref[...] = jnp.dot(xb, wl_ref[...], preferred_element_type=jnp.float32) + bl_ref[...]


def _dense_pre(x, W_src, att_src, W_dst, att_dst, Wl, bl):
    full = lambda s: pl.BlockSpec(s, lambda i: (0, 0))
    blk = lambda s: pl.BlockSpec(s, lambda i: (i, 0))
    return pl.pallas_call(
        _pre_body,
        grid=(GRID,),
        in_specs=[blk((RB, FD)), full((FD, FD)), full((FD, 1)),
                  full((FD, FD)), full((FD, 1)), full((FD, FD)), full((1, FD))],
        out_specs=[blk((RB, FD)), blk((RB, 1)), blk((RB, 1)), blk((RB, FD))],
        out_shape=[jax.ShapeDtypeStruct((NN, FD), jnp.float32),
                   jax.ShapeDtypeStruct((NN, 1), jnp.float32),
                   jax.ShapeDtypeStruct((NN, 1), jnp.float32),
                   jax.ShapeDtypeStruct((NN, FD), jnp.float32)],
    )(x, W_src, att_src.reshape(FD, 1), W_dst, att_dst.reshape(FD, 1),
      Wl, bl.reshape(1, FD))


def _mid_body(n0_ref, n1_ref, d0_ref, d1_ref, b_ref, xl_ref,
              ws_ref, atts_ref, wd_ref, attd_ref, wl_ref, bl_ref,
              xs_ref, as_ref, ad_ref, xl2_ref):
    num = n0_ref[...] + n1_ref[...]
    den = d0_ref[...] + d1_ref[...]
    h = num / (den + EPS) + b_ref[...] + xl_ref[...]
    h = jnp.maximum(h, 0.0)
    xs = jnp.dot(h, ws_ref[...], preferred_element_type=jnp.float32)
    xs_ref[...] = xs
    as_ref[...] = jnp.dot(xs, atts_ref[...], preferred_element_type=jnp.float32)
    wd = jnp.dot(wd_ref[...], attd_ref[...], preferred_element_type=jnp.float32)
    ad_ref[...] = jnp.dot(h, wd, preferred_element_type=jnp.float32)
    xl2_ref[...] = jnp.dot(h, wl_ref[...], preferred_element_type=jnp.float32) + bl_ref[...]


def _dense_mid(num, den, b, xl, W_src, att_src, W_dst, att_dst, Wl, bl):
    full = lambda s: pl.BlockSpec(s, lambda i: (0, 0))
    blk = lambda s: pl.BlockSpec(s, lambda i: (i, 0))
    return pl.pallas_call(
        _mid_body,
        grid=(GRID,),
        in_specs=[blk((RB, FD)), blk((RB, FD)), blk((RB, 1)), blk((RB, 1)),
                  full((1, FD)), blk((RB, FD)),
                  full((FD, FD)), full((FD, 1)), full((FD, FD)), full((FD, 1)),
                  full((FD, FD)), full((1, FD))],
        out_specs=[blk((RB, FD)), blk((RB, 1)), blk((RB, 1)), blk((RB, FD))],
        out_shape=[jax.ShapeDtypeStruct((NN, FD), jnp.float32),
                   jax.ShapeDtypeStruct((NN, 1), jnp.float32),
                   jax.ShapeDtypeStruct((NN, 1), jnp.float32),
                   jax.ShapeDtypeStruct((NN, FD), jnp.float32)],
    )(num[0], num[1], den[0].reshape(NP, 1), den[1].reshape(NP, 1),
      b.reshape(1, FD), xl,
      W_src, att_src.reshape(FD, 1), W_dst, att_dst.reshape(FD, 1),
      Wl, bl.reshape(1, FD))


def _fin_body(n0_ref, n1_ref, d0_ref, d1_ref, b_ref, xl_ref, out_ref):
    num = n0_ref[...] + n1_ref[...]
    den = d0_ref[...] + d1_ref[...]
    out_ref[...] = num / (den + EPS) + b_ref[...] + xl_ref[...]


def _dense_fin(num, den, b, xl):
    full = lambda s: pl.BlockSpec(s, lambda i: (0, 0))
    blk = lambda s: pl.BlockSpec(s, lambda i: (i, 0))
    return pl.pallas_call(
        _fin_body,
        grid=(GRID,),
        in_specs=[blk((RB, FD)), blk((RB, FD)), blk((RB, 1)), blk((RB, 1)),
                  full((1, FD)), blk((RB, FD))],
        out_specs=blk((RB, FD)),
        out_shape=jax.ShapeDtypeStruct((NN, FD), jnp.float32),
    )(num[0], num[1], den[0].reshape(NP, 1), den[1].reshape(NP, 1),
      b.reshape(1, FD), xl)


def _edge_body(xs_hbm, as_hbm, ad_hbm, src_hbm, dst_hbm,
               num_out, den_out,
               src_v, dst_v, rows0_v, rows1_v, e20_v, e21_v,
               asg0_v, asg1_v, adg0_v, adg1_v, zden_v,
               as_sh, ad_sh, acc_num, acc_den,
               semg0, semg1):
    c = lax.axis_index("c")
    s = lax.axis_index("s")
    wid = c * NS + s
    base = s * RPT

    rows = (rows0_v, rows1_v)
    e2b = (e20_v, e21_v)
    asg = (asg0_v, asg1_v)
    adg = (adg0_v, adg1_v)
    semg = (semg0, semg1)

    z16 = jnp.zeros((LN,), jnp.float32)

    @pl.loop(0, CK)
    def _zr(i):
        for cc in range(FD // LN):
            rows0_v[i, pl.ds(cc * LN, LN)] = z16

    @pl.loop(0, RPT // LN)
    def _zd(i):
        zden_v[pl.ds(i * LN, LN)] = z16

    for k in range(RPT // CK):
        pltpu.sync_copy(rows0_v, acc_num.at[pl.ds(base + k * CK, CK)])
    pltpu.sync_copy(zden_v, acc_den.at[pl.ds(base, RPT)])

    @pl.when(s == 0)
    def _stage():
        pltpu.sync_copy(as_hbm, as_sh)
        pltpu.sync_copy(ad_hbm, ad_sh)

    pltpu.sync_copy(src_hbm.at[wid], src_v)
    pltpu.sync_copy(dst_hbm.at[wid], dst_v)

    plsc.subcore_barrier()


    def start_gather(b, ci):
        return pltpu.async_copy(xs_hbm.at[src_v.at[ci]], rows[b], semg[b])

    def gather_scalars(b, ci):
        pltpu.sync_copy(as_sh.at[src_v.at[ci]], asg[b])
        pltpu.sync_copy(ad_sh.at[dst_v.at[ci]], adg[b])

    def scatter(b, ci):
        pltpu.sync_copy(rows[b], acc_num.at[dst_v.at[ci]], add=True)
        pltpu.sync_copy(e2b[b], acc_den.at[dst_v.at[ci]], add=True)

    def process(b, ci):
        e2s = []
        for g in range(CK // LN):
            e = asg[b][pl.ds(g * LN, LN)] + adg[b][pl.ds(g * LN, LN)]
            e = jnp.where(e > 0.0, e, e * 0.2)
            e2 = jnp.exp(e)
            e2b[b][pl.ds(g * LN, LN)] = e2
            e2s.append(e2)
        for g in range(CK // LN):
            ev = e2s[g]
            for j in range(LN):
                sv = ev[j]
                r = g * LN + j
                for cc in range(FD // LN):
                    rows[b][r, pl.ds(cc * LN, LN)] = rows[b][r, pl.ds(cc * LN, LN)] * sv

    @pl.loop(0, NCHUNK - 1, step=2)
    def _chunk(ci):
        d0 = start_gather(0, ci)
        d1 = start_gather(1, ci + 1)

        gather_scalars(0, ci)
        d0.wait()
        process(0, ci)
        scatter(0, ci)

        gather_scalars(1, ci + 1)
        d1.wait()
        process(1, ci + 1)
        scatter(1, ci + 1)

    ti = NCHUNK - 1
    dt = start_gather(0, ti)
    gather_scalars(0, ti)
    dt.wait()
    process(0, ti)
    scatter(0, ti)

    plsc.subcore_barrier()

    pltpu.sync_copy(acc_num.at[pl.ds(base, RPT)], num_out.at[c, pl.ds(base, RPT)])
    pltpu.sync_copy(acc_den.at[pl.ds(base, RPT)], den_out.at[c, pl.ds(base, RPT)])


def _gat_edges(xs, a_s, a_d, src_r, dst_r):
    mesh = plsc.VectorSubcoreMesh(core_axis_name="c", subcore_axis_name="s",
                                  num_cores=NC, num_subcores=NS)
    return pl.kernel(
        _edge_body,
        out_type=(jax.ShapeDtypeStruct((NC, NP, FD), jnp.float32),
                  jax.ShapeDtypeStruct((NC, NP), jnp.float32)),
        mesh=mesh,
        compiler_params=pltpu.CompilerParams(needs_layout_passes=False,
                                             use_tc_tiling_on_sc=False),
        scratch_types=[
            pltpu.VMEM((NCHUNK, CK), jnp.int32),
            pltpu.VMEM((NCHUNK, CK), jnp.int32),
            pltpu.VMEM((CK, FD), jnp.float32),
            pltpu.VMEM((CK, FD), jnp.float32),
            pltpu.VMEM((CK,), jnp.float32),
            pltpu.VMEM((CK,), jnp.float32),
            pltpu.VMEM((CK,), jnp.float32),
            pltpu.VMEM((CK,), jnp.float32),
            pltpu.VMEM((CK,), jnp.float32),
            pltpu.VMEM((CK,), jnp.float32),
            pltpu.VMEM((RPT,), jnp.float32),
            pltpu.VMEM_SHARED((NN,), jnp.float32),
            pltpu.VMEM_SHARED((NN,), jnp.float32),
            pltpu.VMEM_SHARED((NP, FD), jnp.float32),
            pltpu.VMEM_SHARED((NP,), jnp.float32),
            pltpu.SemaphoreType.DMA,
            pltpu.SemaphoreType.DMA,
        ],
    )(xs, a_s, a_d, src_r, dst_r)


def kernel(x, edge_index, W_src1, W_dst1, att_src1, att_dst1, b1, Wl1, bl1,
           W_src2, W_dst2, att_src2, att_dst2, b2, Wl2, bl2):
    src_r = edge_index[0].reshape(NW, NCHUNK, CK)
    dst_r = edge_index[1].reshape(NW, NCHUNK, CK)

    xs1, a_s1, a_d1, xl1 = _dense_pre(x, W_src1, att_src1, W_dst1, att_dst1,
                                      Wl1, bl1)
    num1, den1 = _gat_edges(xs1, a_s1.reshape(NN), a_d1.reshape(NN),
                            src_r, dst_r)
    xs2, a_s2, a_d2, xl2 = _dense_mid(num1, den1, b1, xl1, W_src2, att_src2,
                                      W_dst2, att_dst2, Wl2, bl2)
    num2, den2 = _gat_edges(xs2, a_s2.reshape(NN), a_d2.reshape(NN),
                            src_r, dst_r)
    return _dense_fin(num2, den2, b2, xl2)


# speedup vs baseline: 32.9888x; 1.0139x over previous
import jax
import jax.numpy as jnp
from jax import lax
from jax.experimental import pallas as pl
from jax.experimental.pallas import tpu as pltpu
from jax.experimental.pallas import tpu_sc as plsc

NN = 10000
EE = 320000
FD = 128
NC, NS, LN = 2, 16, 16
NW = NC * NS
EPT = EE // NW
CK = 80
NCHUNK = EPT // CK
NP = 10240
RPT = NP // NS
EPS = 1e-16

RB = 400
GRID = NN // RB


def _pre_body(x_ref, ws_ref, atts_ref, wd_ref, attd_ref, wl_ref, bl_ref,
              xs_ref, as_ref, ad_ref, xl_ref):
    xb = x_ref[...]
    xs = jnp.dot(xb, ws_ref[...], preferred_element_type=jnp.float32)
    xs_ref[...] = xs
    as_ref[...] = jnp.dot(xs, atts_ref[...], preferred_element_type=jnp.float32)
    wd = jnp.dot(wd_ref[...], attd_ref[...], preferred_element_type=jnp.float32)
    ad_ref[...] = jnp.dot(xb, wd, preferred_element_type=jnp.float32)
    xl_ref[...] = jnp.dot(xb, wl_ref[...], preferred_element_type=jnp.float32) + bl_ref[...]


def _dense_pre(x, W_src, att_src, W_dst, att_dst, Wl, bl):
    full = lambda s: pl.BlockSpec(s, lambda i: (0, 0))
    blk = lambda s: pl.BlockSpec(s, lambda i: (i, 0))
    return pl.pallas_call(
        _pre_body,
        grid=(GRID,),
        in_specs=[blk((RB, FD)), full((FD, FD)), full((FD, 1)),
                  full((FD, FD)), full((FD, 1)), full((FD, FD)), full((1, FD))],
        out_specs=[blk((RB, FD)), blk((RB, 1)), blk((RB, 1)), blk((RB, FD))],
        out_shape=[jax.ShapeDtypeStruct((NN, FD), jnp.float32),
                   jax.ShapeDtypeStruct((NN, 1), jnp.float32),
                   jax.ShapeDtypeStruct((NN, 1), jnp.float32),
                   jax.ShapeDtypeStruct((NN, FD), jnp.float32)],
    )(x, W_src, att_src.reshape(FD, 1), W_dst, att_dst.reshape(FD, 1),
      Wl, bl.reshape(1, FD))


def _mid_body(n0_ref, n1_ref, d0_ref, d1_ref, b_ref, xl_ref,
              ws_ref, atts_ref, wd_ref, attd_ref, wl_ref, bl_ref,
              xs_ref, as_ref, ad_ref, xl2_ref):
    num = n0_ref[...] + n1_ref[...]
    den = d0_ref[...] + d1_ref[...]
    h = num / (den + EPS) + b_ref[...] + xl_ref[...]
    h = jnp.maximum(h, 0.0)
    xs = jnp.dot(h, ws_ref[...], preferred_element_type=jnp.float32)
    xs_ref[...] = xs
    as_ref[...] = jnp.dot(xs, atts_ref[...], preferred_element_type=jnp.float32)
    wd = jnp.dot(wd_ref[...], attd_ref[...], preferred_element_type=jnp.float32)
    ad_ref[...] = jnp.dot(h, wd, preferred_element_type=jnp.float32)
    xl2_ref[...] = jnp.dot(h, wl_ref[...], preferred_element_type=jnp.float32) + bl_ref[...]


def _dense_mid(num, den, b, xl, W_src, att_src, W_dst, att_dst, Wl, bl):
    full = lambda s: pl.BlockSpec(s, lambda i: (0, 0))
    blk = lambda s: pl.BlockSpec(s, lambda i: (i, 0))
    return pl.pallas_call(
        _mid_body,
        grid=(GRID,),
        in_specs=[blk((RB, FD)), blk((RB, FD)), blk((RB, 1)), blk((RB, 1)),
                  full((1, FD)), blk((RB, FD)),
                  full((FD, FD)), full((FD, 1)), full((FD, FD)), full((FD, 1)),
                  full((FD, FD)), full((1, FD))],
        out_specs=[blk((RB, FD)), blk((RB, 1)), blk((RB, 1)), blk((RB, FD))],
        out_shape=[jax.ShapeDtypeStruct((NN, FD), jnp.float32),
                   jax.ShapeDtypeStruct((NN, 1), jnp.float32),
                   jax.ShapeDtypeStruct((NN, 1), jnp.float32),
                   jax.ShapeDtypeStruct((NN, FD), jnp.float32)],
    )(num[0], num[1], den[0].reshape(NP, 1), den[1].reshape(NP, 1),
      b.reshape(1, FD), xl,
      W_src, att_src.reshape(FD, 1), W_dst, att_dst.reshape(FD, 1),
      Wl, bl.reshape(1, FD))


def _fin_body(n0_ref, n1_ref, d0_ref, d1_ref, b_ref, xl_ref, out_ref):
    num = n0_ref[...] + n1_ref[...]
    den = d0_ref[...] + d1_ref[...]
    out_ref[...] = num / (den + EPS) + b_ref[...] + xl_ref[...]


def _dense_fin(num, den, b, xl):
    full = lambda s: pl.BlockSpec(s, lambda i: (0, 0))
    blk = lambda s: pl.BlockSpec(s, lambda i: (i, 0))
    return pl.pallas_call(
        _fin_body,
        grid=(GRID,),
        in_specs=[blk((RB, FD)), blk((RB, FD)), blk((RB, 1)), blk((RB, 1)),
                  full((1, FD)), blk((RB, FD))],
        out_specs=blk((RB, FD)),
        out_shape=jax.ShapeDtypeStruct((NN, FD), jnp.float32),
    )(num[0], num[1], den[0].reshape(NP, 1), den[1].reshape(NP, 1),
      b.reshape(1, FD), xl)


def _edge_body(xs_hbm, as_hbm, ad_hbm, src_hbm, dst_hbm,
               num_out, den_out,
               src_v, dst_v, rows0_v, rows1_v, e20_v, e21_v,
               asg0_v, asg1_v, adg0_v, adg1_v, zden_v,
               acc_num, acc_den,
               semg0, semg1):
    c = lax.axis_index("c")
    s = lax.axis_index("s")
    wid = c * NS + s
    base = s * RPT

    rows = (rows0_v, rows1_v)
    e2b = (e20_v, e21_v)
    asg = (asg0_v, asg1_v)
    adg = (adg0_v, adg1_v)
    semg = (semg0, semg1)

    z16 = jnp.zeros((LN,), jnp.float32)

    @pl.loop(0, CK)
    def _zr(i):
        for cc in range(FD // LN):
            rows0_v[i, pl.ds(cc * LN, LN)] = z16

    @pl.loop(0, RPT // LN)
    def _zd(i):
        zden_v[pl.ds(i * LN, LN)] = z16

    for k in range(RPT // CK):
        pltpu.sync_copy(rows0_v, acc_num.at[pl.ds(base + k * CK, CK)])
    pltpu.sync_copy(zden_v, acc_den.at[pl.ds(base, RPT)])

    pltpu.sync_copy(src_hbm.at[wid], src_v)
    pltpu.sync_copy(dst_hbm.at[wid], dst_v)

    plsc.subcore_barrier()


    def start_gather(b, ci):
        return (pltpu.async_copy(as_hbm.at[src_v.at[ci]], asg[b], semg[b]),
                pltpu.async_copy(ad_hbm.at[dst_v.at[ci]], adg[b], semg[b]),
                pltpu.async_copy(xs_hbm.at[src_v.at[ci]], rows[b], semg[b]))

    def scatter(b, ci):
        pltpu.sync_copy(rows[b], acc_num.at[dst_v.at[ci]], add=True)
        pltpu.sync_copy(e2b[b], acc_den.at[dst_v.at[ci]], add=True)

    def process(b, ci):
        e2s = []
        for g in range(CK // LN):
            e = asg[b][pl.ds(g * LN, LN)] + adg[b][pl.ds(g * LN, LN)]
            e = jnp.where(e > 0.0, e, e * 0.2)
            e2 = jnp.exp(e)
            e2b[b][pl.ds(g * LN, LN)] = e2
            e2s.append(e2)
        for g in range(CK // LN):
            ev = e2s[g]
            for j in range(LN):
                sv = ev[j]
                r = g * LN + j
                for cc in range(FD // LN):
                    rows[b][r, pl.ds(cc * LN, LN)] = rows[b][r, pl.ds(cc * LN, LN)] * sv

    @pl.loop(0, NCHUNK - 1, step=2)
    def _chunk(ci):
        d0 = start_gather(0, ci)
        d1 = start_gather(1, ci + 1)

        for d in d0:
            d.wait()
        process(0, ci)
        scatter(0, ci)

        for d in d1:
            d.wait()
        process(1, ci + 1)
        scatter(1, ci + 1)

    ti = NCHUNK - 1
    dt = start_gather(0, ti)
    for d in dt:
        d.wait()
    process(0, ti)
    scatter(0, ti)

    plsc.subcore_barrier()

    pltpu.sync_copy(acc_num.at[pl.ds(base, RPT)], num_out.at[c, pl.ds(base, RPT)])
    pltpu.sync_copy(acc_den.at[pl.ds(base, RPT)], den_out.at[c, pl.ds(base, RPT)])


def _gat_edges(xs, a_s, a_d, src_r, dst_r):
    mesh = plsc.VectorSubcoreMesh(core_axis_name="c", subcore_axis_name="s",
                                  num_cores=NC, num_subcores=NS)
    return pl.kernel(
        _edge_body,
        out_type=(jax.ShapeDtypeStruct((NC, NP, FD), jnp.float32),
                  jax.ShapeDtypeStruct((NC, NP), jnp.float32)),
        mesh=mesh,
        compiler_params=pltpu.CompilerParams(needs_layout_passes=False,
                                             use_tc_tiling_on_sc=False),
        scratch_types=[
            pltpu.VMEM((NCHUNK, CK), jnp.int32),
            pltpu.VMEM((NCHUNK, CK), jnp.int32),
            pltpu.VMEM((CK, FD), jnp.float32),
            pltpu.VMEM((CK, FD), jnp.float32),
            pltpu.VMEM((CK,), jnp.float32),
            pltpu.VMEM((CK,), jnp.float32),
            pltpu.VMEM((CK,), jnp.float32),
            pltpu.VMEM((CK,), jnp.float32),
            pltpu.VMEM((CK,), jnp.float32),
            pltpu.VMEM((CK,), jnp.float32),
            pltpu.VMEM((RPT,), jnp.float32),
            pltpu.VMEM_SHARED((NP, FD), jnp.float32),
            pltpu.VMEM_SHARED((NP,), jnp.float32),
            pltpu.SemaphoreType.DMA,
            pltpu.SemaphoreType.DMA,
        ],
    )(xs, a_s, a_d, src_r, dst_r)


def kernel(x, edge_index, W_src1, W_dst1, att_src1, att_dst1, b1, Wl1, bl1,
           W_src2, W_dst2, att_src2, att_dst2, b2, Wl2, bl2):
    src_r = edge_index[0].reshape(NW, NCHUNK, CK)
    dst_r = edge_index[1].reshape(NW, NCHUNK, CK)

    xs1, a_s1, a_d1, xl1 = _dense_pre(x, W_src1, att_src1, W_dst1, att_dst1,
                                      Wl1, bl1)
    num1, den1 = _gat_edges(xs1, a_s1.reshape(NN), a_d1.reshape(NN),
                            src_r, dst_r)
    xs2, a_s2, a_d2, xl2 = _dense_mid(num1, den1, b1, xl1, W_src2, att_src2,
                                      W_dst2, att_dst2, Wl2, bl2)
    num2, den2 = _gat_edges(xs2, a_s2.reshape(NN), a_d2.reshape(NN),
                            src_r, dst_r)
    return _dense_fin(num2, den2, b2, xl2)


# speedup vs baseline: 35.5911x; 1.0789x over previous
import jax
import jax.numpy as jnp
from jax import lax
from jax.experimental import pallas as pl
from jax.experimental.pallas import tpu as pltpu
from jax.experimental.pallas import tpu_sc as plsc

NN = 10000
EE = 320000
FD = 128
NC, NS, LN = 2, 16, 16
NW = NC * NS
EPT = EE // NW
CK = 80
NCHUNK = EPT // CK
NP = 10240
RPT = NP // NS
EPS = 1e-16

RB = 400
GRID = NN // RB


def _pre_body(x_ref, ws_ref, atts_ref, wd_ref, attd_ref, wl_ref, bl_ref,
              xs_ref, as_ref, ad_ref, xl_ref):
    xb = x_ref[...]
    xs = jnp.dot(xb, ws_ref[...], preferred_element_type=jnp.float32)
    xs_ref[...] = xs
    as_ref[...] = jnp.dot(xs, atts_ref[...], preferred_element_type=jnp.float32)
    wd = jnp.dot(wd_ref[...], attd_ref[...], preferred_element_type=jnp.float32)
    ad_ref[...] = jnp.dot(xb, wd, preferred_element_type=jnp.float32)
    xl_ref[...] = jnp.dot(xb, wl_ref[...], preferred_element_type=jnp.float32) + bl_ref[...]


def _dense_pre(x, W_src, att_src, W_dst, att_dst, Wl, bl):
    full = lambda s: pl.BlockSpec(s, lambda i: (0, 0))
    blk = lambda s: pl.BlockSpec(s, lambda i: (i, 0))
    return pl.pallas_call(
        _pre_body,
        grid=(GRID,),
        in_specs=[blk((RB, FD)), full((FD, FD)), full((FD, 1)),
                  full((FD, FD)), full((FD, 1)), full((FD, FD)), full((1, FD))],
        out_specs=[blk((RB, FD)), blk((RB, 1)), blk((RB, 1)), blk((RB, FD))],
        out_shape=[jax.ShapeDtypeStruct((NN, FD), jnp.float32),
                   jax.ShapeDtypeStruct((NN, 1), jnp.float32),
                   jax.ShapeDtypeStruct((NN, 1), jnp.float32),
                   jax.ShapeDtypeStruct((NN, FD), jnp.float32)],
    )(x, W_src, att_src.reshape(FD, 1), W_dst, att_dst.reshape(FD, 1),
      Wl, bl.reshape(1, FD))


def _mid_body(n0_ref, n1_ref, d0_ref, d1_ref, b_ref, xl_ref,
              ws_ref, atts_ref, wd_ref, attd_ref, wl_ref, bl_ref,
              xs_ref, as_ref, ad_ref, xl2_ref):
    num = n0_ref[...] + n1_ref[...]
    den = d0_ref[...] + d1_ref[...]
    h = num / (den + EPS) + b_ref[...] + xl_ref[...]
    h = jnp.maximum(h, 0.0)
    xs = jnp.dot(h, ws_ref[...], preferred_element_type=jnp.float32)
    xs_ref[...] = xs
    as_ref[...] = jnp.dot(xs, atts_ref[...], preferred_element_type=jnp.float32)
    wd = jnp.dot(wd_ref[...], attd_ref[...], preferred_element_type=jnp.float32)
    ad_ref[...] = jnp.dot(h, wd, preferred_element_type=jnp.float32)
    xl2_ref[...] = jnp.dot(h, wl_ref[...], preferred_element_type=jnp.float32) + bl_ref[...]


def _dense_mid(num, den, b, xl, W_src, att_src, W_dst, att_dst, Wl, bl):
    full = lambda s: pl.BlockSpec(s, lambda i: (0, 0))
    blk = lambda s: pl.BlockSpec(s, lambda i: (i, 0))
    return pl.pallas_call(
        _mid_body,
        grid=(GRID,),
        in_specs=[blk((RB, FD)), blk((RB, FD)), blk((RB, 1)), blk((RB, 1)),
                  full((1, FD)), blk((RB, FD)),
                  full((FD, FD)), full((FD, 1)), full((FD, FD)), full((FD, 1)),
                  full((FD, FD)), full((1, FD))],
        out_specs=[blk((RB, FD)), blk((RB, 1)), blk((RB, 1)), blk((RB, FD))],
        out_shape=[jax.ShapeDtypeStruct((NN, FD), jnp.float32),
                   jax.ShapeDtypeStruct((NN, 1), jnp.float32),
                   jax.ShapeDtypeStruct((NN, 1), jnp.float32),
                   jax.ShapeDtypeStruct((NN, FD), jnp.float32)],
    )(num[0], num[1], den[0].reshape(NP, 1), den[1].reshape(NP, 1),
      b.reshape(1, FD), xl,
      W_src, att_src.reshape(FD, 1), W_dst, att_dst.reshape(FD, 1),
      Wl, bl.reshape(1, FD))


def _fin_body(n0_ref, n1_ref, d0_ref, d1_ref, b_ref, xl_ref, out_ref):
    num = n0_ref[...] + n1_ref[...]
    den = d0_ref[...] + d1_ref[...]
    out_ref[...] = num / (den + EPS) + b_ref[...] + xl_ref[...]


def _dense_fin(num, den, b, xl):
    full = lambda s: pl.BlockSpec(s, lambda i: (0, 0))
    blk = lambda s: pl.BlockSpec(s, lambda i: (i, 0))
    return pl.pallas_call(
        _fin_body,
        grid=(GRID,),
        in_specs=[blk((RB, FD)), blk((RB, FD)), blk((RB, 1)), blk((RB, 1)),
                  full((1, FD)), blk((RB, FD))],
        out_specs=blk((RB, FD)),
        out_shape=jax.ShapeDtypeStruct((NN, FD), jnp.float32),
    )(num[0], num[1], den[0].reshape(NP, 1), den[1].reshape(NP, 1),
      b.reshape(1, FD), xl)


def _edge_body(xs_hbm, as_hbm, ad_hbm, src_hbm, dst_hbm,
               num_out, den_out,
               src_v, dst_v, rows0_v, rows1_v, e20_v, e21_v,
               asg0_v, asg1_v, adg0_v, adg1_v, zden_v,
               acc_num, acc_den,
               semg0, semg1, sems0):
    c = lax.axis_index("c")
    s = lax.axis_index("s")
    wid = c * NS + s
    base = s * RPT

    rows = (rows0_v, rows1_v)
    e2b = (e20_v, e21_v)
    asg = (asg0_v, asg1_v)
    adg = (adg0_v, adg1_v)
    semg = (semg0, semg1)

    z16 = jnp.zeros((LN,), jnp.float32)

    @pl.loop(0, CK)
    def _zr(i):
        for cc in range(FD // LN):
            rows0_v[i, pl.ds(cc * LN, LN)] = z16

    @pl.loop(0, RPT // LN)
    def _zd(i):
        zden_v[pl.ds(i * LN, LN)] = z16

    for k in range(RPT // CK):
        pltpu.sync_copy(rows0_v, acc_num.at[pl.ds(base + k * CK, CK)])
    pltpu.sync_copy(zden_v, acc_den.at[pl.ds(base, RPT)])

    pltpu.sync_copy(src_hbm.at[wid], src_v)
    pltpu.sync_copy(dst_hbm.at[wid], dst_v)

    plsc.subcore_barrier()


    def start_gather(b, ci):
        return (pltpu.async_copy(as_hbm.at[src_v.at[ci]], asg[b], semg[b]),
                pltpu.async_copy(ad_hbm.at[dst_v.at[ci]], adg[b], semg[b]),
                pltpu.async_copy(xs_hbm.at[src_v.at[ci]], rows[b], semg[b]))

    def scatter(b, ci):
        pltpu.sync_copy(rows[b], acc_num.at[dst_v.at[ci]], add=True)
        pltpu.sync_copy(e2b[b], acc_den.at[dst_v.at[ci]], add=True)

    def start_scatter(b, ci, sem):
        return (pltpu.async_copy(rows[b], acc_num.at[dst_v.at[ci]], sem,
                                 add=True),
                pltpu.async_copy(e2b[b], acc_den.at[dst_v.at[ci]], sem,
                                 add=True))

    def process(b, ci):
        e2s = []
        for g in range(CK // LN):
            e = asg[b][pl.ds(g * LN, LN)] + adg[b][pl.ds(g * LN, LN)]
            e = jnp.where(e > 0.0, e, e * 0.2)
            e2 = jnp.exp(e)
            e2b[b][pl.ds(g * LN, LN)] = e2
            e2s.append(e2)
        for g in range(CK // LN):
            ev = e2s[g]
            for j in range(LN):
                sv = ev[j]
                r = g * LN + j
                for cc in range(FD // LN):
                    rows[b][r, pl.ds(cc * LN, LN)] = rows[b][r, pl.ds(cc * LN, LN)] * sv

    @pl.loop(0, NCHUNK - 1, step=2)
    def _chunk(ci):
        d0 = start_gather(0, ci)
        d1 = start_gather(1, ci + 1)

        for d in d0:
            d.wait()
        process(0, ci)
        s0 = start_scatter(0, ci, sems0)

        for d in d1:
            d.wait()
        process(1, ci + 1)
        for d in s0:
            d.wait()
        scatter(1, ci + 1)

    ti = NCHUNK - 1
    dt = start_gather(0, ti)
    for d in dt:
        d.wait()
    process(0, ti)
    scatter(0, ti)

    plsc.subcore_barrier()

    pltpu.sync_copy(acc_num.at[pl.ds(base, RPT)], num_out.at[c, pl.ds(base, RPT)])
    pltpu.sync_copy(acc_den.at[pl.ds(base, RPT)], den_out.at[c, pl.ds(base, RPT)])


def _gat_edges(xs, a_s, a_d, src_r, dst_r):
    mesh = plsc.VectorSubcoreMesh(core_axis_name="c", subcore_axis_name="s",
                                  num_cores=NC, num_subcores=NS)
    return pl.kernel(
        _edge_body,
        out_type=(jax.ShapeDtypeStruct((NC, NP, FD), jnp.float32),
                  jax.ShapeDtypeStruct((NC, NP), jnp.float32)),
        mesh=mesh,
        compiler_params=pltpu.CompilerParams(needs_layout_passes=False,
                                             use_tc_tiling_on_sc=False),
        scratch_types=[
            pltpu.VMEM((NCHUNK, CK), jnp.int32),
            pltpu.VMEM((NCHUNK, CK), jnp.int32),
            pltpu.VMEM((CK, FD), jnp.float32),
            pltpu.VMEM((CK, FD), jnp.float32),
            pltpu.VMEM((CK,), jnp.float32),
            pltpu.VMEM((CK,), jnp.float32),
            pltpu.VMEM((CK,), jnp.float32),
            pltpu.VMEM((CK,), jnp.float32),
            pltpu.VMEM((CK,), jnp.float32),
            pltpu.VMEM((CK,), jnp.float32),
            pltpu.VMEM((RPT,), jnp.float32),
            pltpu.VMEM_SHARED((NP, FD), jnp.float32),
            pltpu.VMEM_SHARED((NP,), jnp.float32),
            pltpu.SemaphoreType.DMA,
            pltpu.SemaphoreType.DMA,
            pltpu.SemaphoreType.DMA,
        ],
    )(xs, a_s, a_d, src_r, dst_r)


def kernel(x, edge_index, W_src1, W_dst1, att_src1, att_dst1, b1, Wl1, bl1,
           W_src2, W_dst2, att_src2, att_dst2, b2, Wl2, bl2):
    src_r = edge_index[0].reshape(NW, NCHUNK, CK)
    dst_r = edge_index[1].reshape(NW, NCHUNK, CK)

    xs1, a_s1, a_d1, xl1 = _dense_pre(x, W_src1, att_src1, W_dst1, att_dst1,
                                      Wl1, bl1)
    num1, den1 = _gat_edges(xs1, a_s1.reshape(NN), a_d1.reshape(NN),
                            src_r, dst_r)
    xs2, a_s2, a_d2, xl2 = _dense_mid(num1, den1, b1, xl1, W_src2, att_src2,
                                      W_dst2, att_dst2, Wl2, bl2)
    num2, den2 = _gat_edges(xs2, a_s2.reshape(NN), a_d2.reshape(NN),
                            src_r, dst_r)
    return _dense_fin(num2, den2, b2, xl2)
